# Initial kernel scaffold; baseline (speedup 1.0000x reference)
import functools

import jax
import jax.numpy as jnp
from jax import lax
from jax.experimental import pallas as pl
from jax.experimental.pallas import tpu as pltpu
from jax.experimental.pallas import tpu_sc as plsc

N = 10000
D = 256
H = 128
E = 160000
NC = 2
NS = 16
CH = 128
EP = 161792
EPT = EP // NS
NCHUNK = EPT // CH
ACC_R = 10240
RPT = N // NS
RB = 1000


def _sc_mesh():
    return plsc.VectorSubcoreMesh(
        core_axis_name="c", subcore_axis_name="s", num_cores=NC, num_subcores=NS
    )


def _deg_body(dst_ref, cnt_ref, acc, dstv, ones, zbuf, sem):
    c = lax.axis_index("c")
    s = lax.axis_index("s")
    z16 = jnp.zeros((16,), jnp.float32)
    o16 = jnp.ones((16,), jnp.float32)
    for r in range(16):
        zbuf[r, pl.ds(0, 16)] = z16
    for r in range(CH):
        ones[r, pl.ds(0, 16)] = o16

    def zloop(k, carry):
        pltpu.sync_copy(zbuf, acc.at[pl.ds(s * 640 + k * 16, 16)])
        return carry

    lax.fori_loop(0, 40, zloop, 0)
    plsc.subcore_barrier()

    base0 = s * EPT

    def eloop(k, carry):
        pltpu.sync_copy(dst_ref.at[pl.ds(base0 + k * CH, CH)], dstv)
        pltpu.sync_copy(ones, acc.at[dstv], add=True)
        return carry

    lax.fori_loop(0, NCHUNK, eloop, 0)
    plsc.subcore_barrier()

    @pl.when(c == 0)
    def _():
        pltpu.sync_copy(acc.at[pl.ds(s * RPT, RPT)], cnt_ref.at[pl.ds(s * RPT, RPT)])


def _deg_count(dst_p):
    k = functools.partial(
        pl.kernel,
        out_type=jax.ShapeDtypeStruct((N, 16), jnp.float32),
        mesh=_sc_mesh(),
        scratch_types=[
            pltpu.MemorySpace.VMEM_SHARED((ACC_R, 16), jnp.float32),
            pltpu.VMEM((CH,), jnp.int32),
            pltpu.VMEM((CH, 16), jnp.float32),
            pltpu.VMEM((16, 16), jnp.float32),
            pltpu.SemaphoreType.DMA,
        ],
    )(_deg_body)
    return k(dst_p)


def _edge_body(y_ref, src_ref, dst_ref, out_ref, acc, srcv, dstv, rows, zbuf, sem):
    c = lax.axis_index("c")
    s = lax.axis_index("s")
    z16 = jnp.zeros((16,), jnp.float32)
    for r in range(16):
        for q in range(8):
            zbuf[r, pl.ds(q * 16, 16)] = z16

    def zloop(k, carry):
        pltpu.sync_copy(zbuf, acc.at[pl.ds(s * 640 + k * 16, 16)])
        return carry

    lax.fori_loop(0, 40, zloop, 0)
    plsc.subcore_barrier()

    base0 = s * EPT
    off = c * N

    def eloop(k, carry):
        b = base0 + k * CH
        pltpu.sync_copy(src_ref.at[pl.ds(b, CH)], srcv)
        pltpu.sync_copy(dst_ref.at[pl.ds(b, CH)], dstv)
        for q in range(CH // 16):
            srcv[pl.ds(q * 16, 16)] = srcv[pl.ds(q * 16, 16)] + off
        pltpu.async_copy(y_ref.at[srcv], rows, sem).wait()
        pltpu.sync_copy(rows, acc.at[dstv], add=True)
        return carry

    lax.fori_loop(0, NCHUNK, eloop, 0)
    plsc.subcore_barrier()

    dst_row = c * N + s * RPT
    pltpu.sync_copy(acc.at[pl.ds(s * RPT, RPT)], out_ref.at[pl.ds(dst_row, RPT)])


def _edge_pass(y2, src_p, dst_p):
    k = functools.partial(
        pl.kernel,
        out_type=jax.ShapeDtypeStruct((NC * N, H), jnp.float32),
        mesh=_sc_mesh(),
        scratch_types=[
            pltpu.MemorySpace.VMEM_SHARED((ACC_R, H), jnp.float32),
            pltpu.VMEM((CH,), jnp.int32),
            pltpu.VMEM((CH,), jnp.int32),
            pltpu.VMEM((CH, H), jnp.float32),
            pltpu.VMEM((16, H), jnp.float32),
            pltpu.SemaphoreType.DMA,
        ],
    )(_edge_body)
    return k(y2, src_p, dst_p)


def _mm1_body(x_ref, w_ref, o_ref):
    o_ref[...] = jnp.dot(x_ref[...], w_ref[...], preferred_element_type=jnp.float32)


def _matmul1(x, W1):
    return pl.pallas_call(
        _mm1_body,
        grid=(N // RB, 2),
        in_specs=[
            pl.BlockSpec((RB, D), lambda i, j: (i, 0)),
            pl.BlockSpec((D, H), lambda i, j: (0, j)),
        ],
        out_specs=pl.BlockSpec((RB, H), lambda i, j: (j * (N // RB) + i, 0)),
        out_shape=jax.ShapeDtypeStruct((NC * N, H), jnp.float32),
    )(x, W1)


def _scale_body(xw_ref, cnt_ref, y_ref):
    dinv = lax.rsqrt(cnt_ref[:, 0:1] + 1.0)
    y_ref[...] = xw_ref[...] * dinv


def _scale(xw2, cnt):
    nb = N // RB
    return pl.pallas_call(
        _scale_body,
        grid=(2 * nb,),
        in_specs=[
            pl.BlockSpec((RB, H), lambda k: (k, 0)),
            pl.BlockSpec((RB, 16), lambda k: (k % nb, 0)),
        ],
        out_specs=pl.BlockSpec((RB, H), lambda k: (k, 0)),
        out_shape=jax.ShapeDtypeStruct((NC * N, H), jnp.float32),
    )(xw2, cnt)


def _layer2_body(a0_ref, a1_ref, y0_ref, y1_ref, cnt_ref, b1_ref, w_ref, o_ref):
    dinv = lax.rsqrt(cnt_ref[:, 0:1] + 1.0)
    h0 = jnp.maximum((a0_ref[...] + y0_ref[...]) * dinv + b1_ref[0, 0:1, :], 0.0)
    h1 = jnp.maximum((a1_ref[...] + y1_ref[...]) * dinv + b1_ref[1, 0:1, :], 0.0)
    z = jnp.dot(h0, w_ref[0:H, :], preferred_element_type=jnp.float32)
    z = z + jnp.dot(h1, w_ref[H:D, :], preferred_element_type=jnp.float32)
    o_ref[...] = z * dinv


def _layer2(acc1, y1, cnt, b1p, W2):
    nb = N // RB
    return pl.pallas_call(
        _layer2_body,
        grid=(nb, 2),
        in_specs=[
            pl.BlockSpec((RB, H), lambda i, j: (i, 0)),
            pl.BlockSpec((RB, H), lambda i, j: (nb + i, 0)),
            pl.BlockSpec((RB, H), lambda i, j: (i, 0)),
            pl.BlockSpec((RB, H), lambda i, j: (nb + i, 0)),
            pl.BlockSpec((RB, 16), lambda i, j: (i, 0)),
            pl.BlockSpec((2, 8, H), lambda i, j: (0, 0, 0)),
            pl.BlockSpec((D, H), lambda i, j: (0, j)),
        ],
        out_specs=pl.BlockSpec((RB, H), lambda i, j: (j * nb + i, 0)),
        out_shape=jax.ShapeDtypeStruct((NC * N, H), jnp.float32),
    )(acc1, acc1, y1, y1, cnt, b1p, W2)


def _final_body(a_ref, y_ref, cnt_ref, b2_ref, o_ref):
    j = pl.program_id(1)
    dinv = lax.rsqrt(cnt_ref[:, 0:1] + 1.0)
    o_ref[...] = (a_ref[...] + y_ref[...]) * dinv + b2_ref[j, 0:1, :]


def _final(acc2, yz, cnt, b2p):
    nb = N // RB
    return pl.pallas_call(
        _final_body,
        grid=(nb, 2),
        in_specs=[
            pl.BlockSpec((RB, H), lambda i, j: (j * nb + i, 0)),
            pl.BlockSpec((RB, H), lambda i, j: (j * nb + i, 0)),
            pl.BlockSpec((RB, 16), lambda i, j: (i, 0)),
            pl.BlockSpec((2, 8, H), lambda i, j: (0, 0, 0)),
        ],
        out_specs=pl.BlockSpec((RB, H), lambda i, j: (i, j)),
        out_shape=jax.ShapeDtypeStruct((N, D), jnp.float32),
    )(acc2, yz, cnt, b2p)


def kernel(x, edge_index, W1, b1, W2, b2):
    ei = edge_index.astype(jnp.int32)
    pad = EP - E
    src_p = jnp.concatenate([ei[0], jnp.zeros((pad,), jnp.int32)])
    dst_p = jnp.concatenate([ei[1], jnp.full((pad,), N, jnp.int32)])
    b1p = jnp.broadcast_to(b1.reshape(2, 1, H), (2, 8, H))
    b2p = jnp.broadcast_to(b2.reshape(2, 1, H), (2, 8, H))

    cnt = _deg_count(dst_p)
    xw2 = _matmul1(x, W1)
    y1 = _scale(xw2, cnt)
    acc1 = _edge_pass(y1, src_p, dst_p)
    yz = _layer2(acc1, y1, cnt, b1p, W2)
    acc2 = _edge_pass(yz, src_p, dst_p)
    return _final(acc2, yz, cnt, b2p)


# speedup vs baseline: 7.7305x; 7.7305x over previous
import functools

import jax
import jax.numpy as jnp
from jax import lax
from jax.experimental import pallas as pl
from jax.experimental.pallas import tpu as pltpu
from jax.experimental.pallas import tpu_sc as plsc

N = 10000
D = 256
H = 128
E = 160000
NC = 2
NS = 16
CH = 128
EP = 161792
EPT = EP // NS
NCHUNK = EPT // CH
ACC_R = 10240
RPT = N // NS
RB = 1000


def _sc_mesh():
    return plsc.VectorSubcoreMesh(
        core_axis_name="c", subcore_axis_name="s", num_cores=NC, num_subcores=NS
    )


def _deg_body(dst_ref, cnt_ref, acc, dstv, ones, zbuf, sem):
    c = lax.axis_index("c")
    s = lax.axis_index("s")
    z16 = jnp.zeros((16,), jnp.float32)
    o16 = jnp.ones((16,), jnp.float32)
    for r in range(16):
        zbuf[r, pl.ds(0, 16)] = z16
    for r in range(CH):
        ones[r, pl.ds(0, 16)] = o16

    def zloop(k, carry):
        pltpu.sync_copy(zbuf, acc.at[pl.ds(s * 640 + k * 16, 16)])
        return carry

    lax.fori_loop(0, 40, zloop, 0)
    plsc.subcore_barrier()

    base0 = s * EPT

    def eloop(k, carry):
        pltpu.sync_copy(dst_ref.at[pl.ds(base0 + k * CH, CH)], dstv)
        pltpu.sync_copy(ones, acc.at[dstv], add=True)
        return carry

    lax.fori_loop(0, NCHUNK, eloop, 0)
    plsc.subcore_barrier()

    @pl.when(c == 0)
    def _():
        off_r = pl.multiple_of(jnp.where(s == NS - 1, N - 640, s * 640), 8)
        pltpu.sync_copy(acc.at[pl.ds(off_r, 640)], cnt_ref.at[pl.ds(off_r, 640)])


def _deg_count(dst_p):
    k = functools.partial(
        pl.kernel,
        out_type=jax.ShapeDtypeStruct((N, 16), jnp.float32),
        mesh=_sc_mesh(),
        scratch_types=[
            pltpu.MemorySpace.VMEM_SHARED((ACC_R, 16), jnp.float32),
            pltpu.VMEM((CH,), jnp.int32),
            pltpu.VMEM((CH, 16), jnp.float32),
            pltpu.VMEM((16, 16), jnp.float32),
            pltpu.SemaphoreType.DMA,
        ],
    )(_deg_body)
    return k(dst_p)


def _edge_body(y_ref, src_ref, dst_ref, out_ref, acc, srcv, dstv, rows, zbuf, sem):
    c = lax.axis_index("c")
    s = lax.axis_index("s")
    z16 = jnp.zeros((16,), jnp.float32)
    for r in range(16):
        for q in range(8):
            zbuf[r, pl.ds(q * 16, 16)] = z16

    def zloop(k, carry):
        pltpu.sync_copy(zbuf, acc.at[pl.ds(s * 640 + k * 16, 16)])
        return carry

    lax.fori_loop(0, 40, zloop, 0)
    plsc.subcore_barrier()

    base0 = s * EPT
    off = c * N

    def eloop(k, carry):
        b = base0 + k * CH
        pltpu.sync_copy(src_ref.at[pl.ds(b, CH)], srcv)
        pltpu.sync_copy(dst_ref.at[pl.ds(b, CH)], dstv)
        for q in range(CH // 16):
            srcv[pl.ds(q * 16, 16)] = srcv[pl.ds(q * 16, 16)] + off
        pltpu.async_copy(y_ref.at[srcv], rows, sem).wait()
        pltpu.sync_copy(rows, acc.at[dstv], add=True)
        return carry

    lax.fori_loop(0, NCHUNK, eloop, 0)
    plsc.subcore_barrier()

    off_r = pl.multiple_of(jnp.where(s == NS - 1, N - 640, s * 640), 8)
    dst_row = pl.multiple_of(c * N + off_r, 8)
    pltpu.sync_copy(acc.at[pl.ds(off_r, 640)], out_ref.at[pl.ds(dst_row, 640)])


def _edge_pass(y2, src_p, dst_p):
    k = functools.partial(
        pl.kernel,
        out_type=jax.ShapeDtypeStruct((NC * N, H), jnp.float32),
        mesh=_sc_mesh(),
        scratch_types=[
            pltpu.MemorySpace.VMEM_SHARED((ACC_R, H), jnp.float32),
            pltpu.VMEM((CH,), jnp.int32),
            pltpu.VMEM((CH,), jnp.int32),
            pltpu.VMEM((CH, H), jnp.float32),
            pltpu.VMEM((16, H), jnp.float32),
            pltpu.SemaphoreType.DMA,
        ],
    )(_edge_body)
    return k(y2, src_p, dst_p)


def _mm1_body(x_ref, w_ref, o_ref):
    o_ref[...] = jnp.dot(x_ref[...], w_ref[...], preferred_element_type=jnp.float32)


def _matmul1(x, W1):
    return pl.pallas_call(
        _mm1_body,
        grid=(N // RB, 2),
        in_specs=[
            pl.BlockSpec((RB, D), lambda i, j: (i, 0)),
            pl.BlockSpec((D, H), lambda i, j: (0, j)),
        ],
        out_specs=pl.BlockSpec((RB, H), lambda i, j: (j * (N // RB) + i, 0)),
        out_shape=jax.ShapeDtypeStruct((NC * N, H), jnp.float32),
    )(x, W1)


def _scale_body(xw_ref, cnt_ref, y_ref):
    dinv = lax.rsqrt(cnt_ref[:, 0:1] + 1.0)
    y_ref[...] = xw_ref[...] * dinv


def _scale(xw2, cnt):
    nb = N // RB
    return pl.pallas_call(
        _scale_body,
        grid=(2 * nb,),
        in_specs=[
            pl.BlockSpec((RB, H), lambda k: (k, 0)),
            pl.BlockSpec((RB, 16), lambda k: (k % nb, 0)),
        ],
        out_specs=pl.BlockSpec((RB, H), lambda k: (k, 0)),
        out_shape=jax.ShapeDtypeStruct((NC * N, H), jnp.float32),
    )(xw2, cnt)


def _layer2_body(a0_ref, a1_ref, y0_ref, y1_ref, cnt_ref, b1_ref, w_ref, o_ref):
    dinv = lax.rsqrt(cnt_ref[:, 0:1] + 1.0)
    h0 = jnp.maximum((a0_ref[...] + y0_ref[...]) * dinv + b1_ref[0, 0:1, :], 0.0)
    h1 = jnp.maximum((a1_ref[...] + y1_ref[...]) * dinv + b1_ref[1, 0:1, :], 0.0)
    z = jnp.dot(h0, w_ref[0:H, :], preferred_element_type=jnp.float32)
    z = z + jnp.dot(h1, w_ref[H:D, :], preferred_element_type=jnp.float32)
    o_ref[...] = z * dinv


def _layer2(acc1, y1, cnt, b1p, W2):
    nb = N // RB
    return pl.pallas_call(
        _layer2_body,
        grid=(nb, 2),
        in_specs=[
            pl.BlockSpec((RB, H), lambda i, j: (i, 0)),
            pl.BlockSpec((RB, H), lambda i, j: (nb + i, 0)),
            pl.BlockSpec((RB, H), lambda i, j: (i, 0)),
            pl.BlockSpec((RB, H), lambda i, j: (nb + i, 0)),
            pl.BlockSpec((RB, 16), lambda i, j: (i, 0)),
            pl.BlockSpec((2, 8, H), lambda i, j: (0, 0, 0)),
            pl.BlockSpec((D, H), lambda i, j: (0, j)),
        ],
        out_specs=pl.BlockSpec((RB, H), lambda i, j: (j * nb + i, 0)),
        out_shape=jax.ShapeDtypeStruct((NC * N, H), jnp.float32),
    )(acc1, acc1, y1, y1, cnt, b1p, W2)


def _final_body(a_ref, y_ref, cnt_ref, b2_ref, o_ref):
    j = pl.program_id(1)
    dinv = lax.rsqrt(cnt_ref[:, 0:1] + 1.0)
    o_ref[...] = (a_ref[...] + y_ref[...]) * dinv + b2_ref[j, 0:1, :]


def _final(acc2, yz, cnt, b2p):
    nb = N // RB
    return pl.pallas_call(
        _final_body,
        grid=(nb, 2),
        in_specs=[
            pl.BlockSpec((RB, H), lambda i, j: (j * nb + i, 0)),
            pl.BlockSpec((RB, H), lambda i, j: (j * nb + i, 0)),
            pl.BlockSpec((RB, 16), lambda i, j: (i, 0)),
            pl.BlockSpec((2, 8, H), lambda i, j: (0, 0, 0)),
        ],
        out_specs=pl.BlockSpec((RB, H), lambda i, j: (i, j)),
        out_shape=jax.ShapeDtypeStruct((N, D), jnp.float32),
    )(acc2, yz, cnt, b2p)


def kernel(x, edge_index, W1, b1, W2, b2):
    ei = edge_index.astype(jnp.int32)
    pad = EP - E
    src_p = jnp.concatenate([ei[0], jnp.zeros((pad,), jnp.int32)])
    dst_p = jnp.concatenate([ei[1], jnp.full((pad,), N, jnp.int32)])
    b1p = jnp.broadcast_to(b1.reshape(2, 1, H), (2, 8, H))
    b2p = jnp.broadcast_to(b2.reshape(2, 1, H), (2, 8, H))

    cnt = _deg_count(dst_p)
    xw2 = _matmul1(x, W1)
    y1 = _scale(xw2, cnt)
    acc1 = _edge_pass(y1, src_p, dst_p)
    yz = _layer2(acc1, y1, cnt, b1p, W2)
    acc2 = _edge_pass(yz, src_p, dst_p)
    return _final(acc2, yz, cnt, b2p)


# speedup vs baseline: 7.8633x; 1.0172x over previous
import functools

import jax
import jax.numpy as jnp
from jax import lax
from jax.experimental import pallas as pl
from jax.experimental.pallas import tpu as pltpu
from jax.experimental.pallas import tpu_sc as plsc

N = 10000
D = 256
H = 128
E = 160000
NC = 2
NS = 16
CH = 128
NCHUNK = 80
EPT = NCHUNK * CH
EP = NS * EPT
NBUF = 2
ACC_R = 10240
RPT = N // NS
RB = 1000


def _sc_mesh():
    return plsc.VectorSubcoreMesh(
        core_axis_name="c", subcore_axis_name="s", num_cores=NC, num_subcores=NS
    )


def _deg_body(dst_ref, cnt_ref, acc, dstv, ones, zbuf):
    c = lax.axis_index("c")
    s = lax.axis_index("s")
    hc = NCHUNK // NC
    z16 = jnp.zeros((16,), jnp.float32)
    o16 = jnp.ones((16,), jnp.float32)
    for r in range(16):
        zbuf[r, pl.ds(0, 16)] = z16
    for r in range(CH):
        ones[r, pl.ds(0, 16)] = o16

    def zloop(k, carry):
        pltpu.sync_copy(zbuf, acc.at[pl.ds(s * 640 + k * 16, 16)])
        return carry

    lax.fori_loop(0, 40, zloop, 0)
    plsc.subcore_barrier()

    base = s * EPT + c * hc * CH

    def eloop(k, carry):
        pltpu.sync_copy(dst_ref.at[pl.ds(base + k * CH, CH)], dstv)
        pltpu.sync_copy(ones, acc.at[dstv], add=True)
        return carry

    lax.fori_loop(0, hc, eloop, 0)
    plsc.subcore_barrier()

    off_r = pl.multiple_of(jnp.where(s == NS - 1, N - 640, s * 640), 8)
    dst_row = pl.multiple_of(c * N + off_r, 8)
    pltpu.sync_copy(acc.at[pl.ds(off_r, 640)], cnt_ref.at[pl.ds(dst_row, 640)])


def _deg_count(dst_p):
    k = functools.partial(
        pl.kernel,
        out_type=jax.ShapeDtypeStruct((NC * N, 16), jnp.float32),
        mesh=_sc_mesh(),
        scratch_types=[
            pltpu.MemorySpace.VMEM_SHARED((ACC_R, 16), jnp.float32),
            pltpu.VMEM((CH,), jnp.int32),
            pltpu.VMEM((CH, 16), jnp.float32),
            pltpu.VMEM((16, 16), jnp.float32),
        ],
    )(_deg_body)
    return k(dst_p)


def _edge_body(y_ref, src_ref, dst_ref, out_ref, acc, srcb,
               d0, d1, d2, d3, r0, r1, sd0, sd1, sd2, sd3, sg0, sg1):
    c = lax.axis_index("c")
    s = lax.axis_index("s")
    dstv = (d0, d1, d2, d3)
    dsem = (sd0, sd1, sd2, sd3)
    rows = (r0, r1)
    gsem = (sg0, sg1)
    base = s * EPT

    def load_dst(cidx, slot):
        pltpu.async_copy(
            dst_ref.at[pl.ds(base + cidx * CH, CH)], dstv[slot], dsem[slot]
        )

    def wait_dst(slot):
        pltpu.make_async_copy(dst_ref.at[pl.ds(0, CH)], dstv[slot], dsem[slot]).wait()

    def start_gather(cidx, g):
        pltpu.async_copy(y_ref.at[srcb.at[pl.ds(cidx * CH, CH)]], rows[g], gsem[g])

    def wait_gather(cidx, g):
        pltpu.make_async_copy(
            y_ref.at[srcb.at[pl.ds(cidx * CH, CH)]], rows[g], gsem[g]
        ).wait()

    for b in range(2 * NBUF):
        load_dst(b, b)
    pltpu.sync_copy(src_ref.at[pl.ds(base, EPT)], srcb)

    @pl.when(c == 1)
    def _():
        def aloop(i, carry):
            srcb[pl.ds(i * 16, 16)] = srcb[pl.ds(i * 16, 16)] + N
            return carry

        lax.fori_loop(0, EPT // 16, aloop, 0)

    z16 = jnp.zeros((16,), jnp.float32)
    for r in range(CH):
        for q in range(H // 16):
            r0[r, pl.ds(q * 16, 16)] = z16

    def zloop(k, carry):
        pltpu.sync_copy(r0, acc.at[pl.ds(s * 640 + k * CH, CH)])
        return carry

    lax.fori_loop(0, 640 // CH, zloop, 0)

    for b in range(NBUF):
        start_gather(b, b)
    plsc.subcore_barrier()

    def eloop(k, carry):
        for b in range(2 * NBUF):
            cidx = k * (2 * NBUF) + b
            g = b % NBUF
            wait_gather(cidx, g)
            wait_dst(b)
            pltpu.sync_copy(rows[g], acc.at[dstv[b]], add=True)
            nxt2 = cidx + NBUF

            @pl.when(nxt2 < NCHUNK)
            def _():
                start_gather(nxt2, g)

            nxt4 = cidx + 2 * NBUF

            @pl.when(nxt4 < NCHUNK)
            def _():
                load_dst(nxt4, b)
        return carry

    lax.fori_loop(0, NCHUNK // (2 * NBUF), eloop, 0)
    plsc.subcore_barrier()

    off_r = pl.multiple_of(jnp.where(s == NS - 1, N - 640, s * 640), 8)
    dst_row = pl.multiple_of(c * N + off_r, 8)
    pltpu.sync_copy(acc.at[pl.ds(off_r, 640)], out_ref.at[pl.ds(dst_row, 640)])


def _edge_pass(y2, src_p, dst_p):
    k = functools.partial(
        pl.kernel,
        out_type=jax.ShapeDtypeStruct((NC * N, H), jnp.float32),
        mesh=_sc_mesh(),
        scratch_types=[
            pltpu.MemorySpace.VMEM_SHARED((ACC_R, H), jnp.float32),
            pltpu.VMEM((EPT,), jnp.int32),
            pltpu.VMEM((CH,), jnp.int32),
            pltpu.VMEM((CH,), jnp.int32),
            pltpu.VMEM((CH,), jnp.int32),
            pltpu.VMEM((CH,), jnp.int32),
            pltpu.VMEM((CH, H), jnp.float32),
            pltpu.VMEM((CH, H), jnp.float32),
            pltpu.SemaphoreType.DMA,
            pltpu.SemaphoreType.DMA,
            pltpu.SemaphoreType.DMA,
            pltpu.SemaphoreType.DMA,
            pltpu.SemaphoreType.DMA,
            pltpu.SemaphoreType.DMA,
        ],
    )(_edge_body)
    return k(y2, src_p, dst_p)


def _mm1_body(x_ref, w_ref, o_ref):
    o_ref[...] = jnp.dot(x_ref[...], w_ref[...], preferred_element_type=jnp.float32)


def _matmul1(x, W1):
    return pl.pallas_call(
        _mm1_body,
        grid=(N // RB, 2),
        in_specs=[
            pl.BlockSpec((RB, D), lambda i, j: (i, 0)),
            pl.BlockSpec((D, H), lambda i, j: (0, j)),
        ],
        out_specs=pl.BlockSpec((RB, H), lambda i, j: (j * (N // RB) + i, 0)),
        out_shape=jax.ShapeDtypeStruct((NC * N, H), jnp.float32),
    )(x, W1)


def _scale_body(xw_ref, c0_ref, c1_ref, y_ref, dinv_ref):
    dinv = lax.rsqrt(c0_ref[:, 0:1] + c1_ref[:, 0:1] + 1.0)
    y_ref[...] = xw_ref[...] * dinv
    dinv_ref[...] = jnp.broadcast_to(dinv, (RB, 16))


def _scale(xw2, cnt):
    nb = N // RB
    return pl.pallas_call(
        _scale_body,
        grid=(2 * nb,),
        in_specs=[
            pl.BlockSpec((RB, H), lambda k: (k, 0)),
            pl.BlockSpec((RB, 16), lambda k: (k % nb, 0)),
            pl.BlockSpec((RB, 16), lambda k: (nb + k % nb, 0)),
        ],
        out_specs=[
            pl.BlockSpec((RB, H), lambda k: (k, 0)),
            pl.BlockSpec((RB, 16), lambda k: (k % nb, 0)),
        ],
        out_shape=[
            jax.ShapeDtypeStruct((NC * N, H), jnp.float32),
            jax.ShapeDtypeStruct((N, 16), jnp.float32),
        ],
    )(xw2, cnt, cnt)


def _layer2_body(a0_ref, a1_ref, y0_ref, y1_ref, cnt_ref, b1_ref, w_ref, o_ref):
    dinv = cnt_ref[:, 0:1]
    h0 = jnp.maximum((a0_ref[...] + y0_ref[...]) * dinv + b1_ref[0, 0:1, :], 0.0)
    h1 = jnp.maximum((a1_ref[...] + y1_ref[...]) * dinv + b1_ref[1, 0:1, :], 0.0)
    z = jnp.dot(h0, w_ref[0:H, :], preferred_element_type=jnp.float32)
    z = z + jnp.dot(h1, w_ref[H:D, :], preferred_element_type=jnp.float32)
    o_ref[...] = z * dinv


def _layer2(acc1, y1, cnt, b1p, W2):
    nb = N // RB
    return pl.pallas_call(
        _layer2_body,
        grid=(nb, 2),
        in_specs=[
            pl.BlockSpec((RB, H), lambda i, j: (i, 0)),
            pl.BlockSpec((RB, H), lambda i, j: (nb + i, 0)),
            pl.BlockSpec((RB, H), lambda i, j: (i, 0)),
            pl.BlockSpec((RB, H), lambda i, j: (nb + i, 0)),
            pl.BlockSpec((RB, 16), lambda i, j: (i, 0)),
            pl.BlockSpec((2, 8, H), lambda i, j: (0, 0, 0)),
            pl.BlockSpec((D, H), lambda i, j: (0, j)),
        ],
        out_specs=pl.BlockSpec((RB, H), lambda i, j: (j * nb + i, 0)),
        out_shape=jax.ShapeDtypeStruct((NC * N, H), jnp.float32),
    )(acc1, acc1, y1, y1, cnt, b1p, W2)


def _final_body(a_ref, y_ref, cnt_ref, b2_ref, o_ref):
    j = pl.program_id(1)
    dinv = cnt_ref[:, 0:1]
    o_ref[...] = (a_ref[...] + y_ref[...]) * dinv + b2_ref[j, 0:1, :]


def _final(acc2, yz, cnt, b2p):
    nb = N // RB
    return pl.pallas_call(
        _final_body,
        grid=(nb, 2),
        in_specs=[
            pl.BlockSpec((RB, H), lambda i, j: (j * nb + i, 0)),
            pl.BlockSpec((RB, H), lambda i, j: (j * nb + i, 0)),
            pl.BlockSpec((RB, 16), lambda i, j: (i, 0)),
            pl.BlockSpec((2, 8, H), lambda i, j: (0, 0, 0)),
        ],
        out_specs=pl.BlockSpec((RB, H), lambda i, j: (i, j)),
        out_shape=jax.ShapeDtypeStruct((N, D), jnp.float32),
    )(acc2, yz, cnt, b2p)


def kernel(x, edge_index, W1, b1, W2, b2):
    ei = edge_index.astype(jnp.int32)
    pad = EP - E
    src_p = jnp.concatenate([ei[0], jnp.zeros((pad,), jnp.int32)])
    dst_p = jnp.concatenate([ei[1], jnp.full((pad,), N, jnp.int32)])
    b1p = jnp.broadcast_to(b1.reshape(2, 1, H), (2, 8, H))
    b2p = jnp.broadcast_to(b2.reshape(2, 1, H), (2, 8, H))

    cnt = _deg_count(dst_p)
    xw2 = _matmul1(x, W1)
    y1, dinv16 = _scale(xw2, cnt)
    acc1 = _edge_pass(y1, src_p, dst_p)
    yz = _layer2(acc1, y1, dinv16, b1p, W2)
    acc2 = _edge_pass(yz, src_p, dst_p)
    return _final(acc2, yz, dinv16, b2p)


# speedup vs baseline: 8.0533x; 1.0242x over previous
import functools

import jax
import jax.numpy as jnp
from jax import lax
from jax.experimental import pallas as pl
from jax.experimental.pallas import tpu as pltpu
from jax.experimental.pallas import tpu_sc as plsc

N = 10000
D = 256
H = 128
E = 160000
NC = 2
NS = 16
CH = 128
NCHUNK = 80
EPT = NCHUNK * CH
EP = NS * EPT
NBUF = 2
ACC_R = 10240
RPT = N // NS
RB = 1000


def _sc_mesh():
    return plsc.VectorSubcoreMesh(
        core_axis_name="c", subcore_axis_name="s", num_cores=NC, num_subcores=NS
    )


def _deg_body(dst_ref, cnt_ref, acc, dstv, ones, zbuf):
    c = lax.axis_index("c")
    s = lax.axis_index("s")
    hc = NCHUNK // NC
    z16 = jnp.zeros((16,), jnp.float32)
    o16 = jnp.ones((16,), jnp.float32)
    for r in range(16):
        zbuf[r, pl.ds(0, 16)] = z16
    for r in range(CH):
        ones[r, pl.ds(0, 16)] = o16

    def zloop(k, carry):
        pltpu.sync_copy(zbuf, acc.at[pl.ds(s * 640 + k * 16, 16)])
        return carry

    lax.fori_loop(0, 40, zloop, 0)
    plsc.subcore_barrier()

    base = s * EPT + c * hc * CH

    def eloop(k, carry):
        pltpu.sync_copy(dst_ref.at[pl.ds(base + k * CH, CH)], dstv)
        pltpu.sync_copy(ones, acc.at[dstv], add=True)
        return carry

    lax.fori_loop(0, hc, eloop, 0)
    plsc.subcore_barrier()

    off_r = pl.multiple_of(jnp.where(s == NS - 1, N - 640, s * 640), 8)
    dst_row = pl.multiple_of(c * N + off_r, 8)
    pltpu.sync_copy(acc.at[pl.ds(off_r, 640)], cnt_ref.at[pl.ds(dst_row, 640)])


def _deg_count(dst_p):
    k = functools.partial(
        pl.kernel,
        out_type=jax.ShapeDtypeStruct((NC * N, 16), jnp.float32),
        mesh=_sc_mesh(),
        scratch_types=[
            pltpu.MemorySpace.VMEM_SHARED((ACC_R, 16), jnp.float32),
            pltpu.VMEM((CH,), jnp.int32),
            pltpu.VMEM((CH, 16), jnp.float32),
            pltpu.VMEM((16, 16), jnp.float32),
        ],
    )(_deg_body)
    return k(dst_p)


def _edge_body(y_ref, src_ref, dst_ref, out_ref, acc, srcb,
               d0, d1, d2, d3, r0, r1, sd0, sd1, sd2, sd3, sg0, sg1):
    c = lax.axis_index("c")
    s = lax.axis_index("s")
    dstv = (d0, d1, d2, d3)
    dsem = (sd0, sd1, sd2, sd3)
    rows = (r0, r1)
    gsem = (sg0, sg1)
    base = s * EPT

    def load_dst(cidx, slot):
        pltpu.async_copy(
            dst_ref.at[pl.ds(base + cidx * CH, CH)], dstv[slot], dsem[slot]
        )

    def wait_dst(slot):
        pltpu.make_async_copy(dst_ref.at[pl.ds(0, CH)], dstv[slot], dsem[slot]).wait()

    def start_gather(cidx, g):
        pltpu.async_copy(y_ref.at[srcb.at[pl.ds(cidx * CH, CH)]], rows[g], gsem[g])

    def wait_gather(cidx, g):
        pltpu.make_async_copy(
            y_ref.at[srcb.at[pl.ds(cidx * CH, CH)]], rows[g], gsem[g]
        ).wait()

    for b in range(2 * NBUF):
        load_dst(b, b)
    pltpu.sync_copy(src_ref.at[pl.ds(base, EPT)], srcb)

    @pl.when(c == 1)
    def _():
        def aloop(i, carry):
            srcb[pl.ds(i * 16, 16)] = srcb[pl.ds(i * 16, 16)] + N
            return carry

        lax.fori_loop(0, EPT // 16, aloop, 0)

    z16 = jnp.zeros((16,), jnp.float32)
    for r in range(CH):
        for q in range(H // 16):
            r0[r, pl.ds(q * 16, 16)] = z16

    def zloop(k, carry):
        pltpu.sync_copy(r0, acc.at[pl.ds(s * 640 + k * CH, CH)])
        return carry

    lax.fori_loop(0, 640 // CH, zloop, 0)

    for b in range(NBUF):
        start_gather(b, b)
    plsc.subcore_barrier()

    def eloop(k, carry):
        for b in range(2 * NBUF):
            cidx = k * (2 * NBUF) + b
            g = b % NBUF
            wait_gather(cidx, g)
            wait_dst(b)
            pltpu.sync_copy(rows[g], acc.at[dstv[b]], add=True)
            nxt2 = cidx + NBUF

            @pl.when(nxt2 < NCHUNK)
            def _():
                start_gather(nxt2, g)

            nxt4 = cidx + 2 * NBUF

            @pl.when(nxt4 < NCHUNK)
            def _():
                load_dst(nxt4, b)
        return carry

    lax.fori_loop(0, NCHUNK // (2 * NBUF), eloop, 0)
    plsc.subcore_barrier()

    off_r = pl.multiple_of(jnp.where(s == NS - 1, N - 640, s * 640), 8)
    dst_row = pl.multiple_of(c * N + off_r, 8)
    pltpu.sync_copy(acc.at[pl.ds(off_r, 640)], out_ref.at[pl.ds(dst_row, 640)])


def _edge_pass(y2, src_p, dst_p):
    k = functools.partial(
        pl.kernel,
        out_type=jax.ShapeDtypeStruct((NC * N, H), jnp.float32),
        mesh=_sc_mesh(),
        scratch_types=[
            pltpu.MemorySpace.VMEM_SHARED((ACC_R, H), jnp.float32),
            pltpu.VMEM((EPT,), jnp.int32),
            pltpu.VMEM((CH,), jnp.int32),
            pltpu.VMEM((CH,), jnp.int32),
            pltpu.VMEM((CH,), jnp.int32),
            pltpu.VMEM((CH,), jnp.int32),
            pltpu.VMEM((CH, H), jnp.float32),
            pltpu.VMEM((CH, H), jnp.float32),
            pltpu.SemaphoreType.DMA,
            pltpu.SemaphoreType.DMA,
            pltpu.SemaphoreType.DMA,
            pltpu.SemaphoreType.DMA,
            pltpu.SemaphoreType.DMA,
            pltpu.SemaphoreType.DMA,
        ],
    )(_edge_body)
    return k(y2, src_p, dst_p)


def _mm1_body(x_ref, w_ref, c0_ref, c1_ref, y_ref, dinv_ref):
    dinv = lax.rsqrt(c0_ref[:, 0:1] + c1_ref[:, 0:1] + 1.0)
    xw = jnp.dot(x_ref[...], w_ref[...], preferred_element_type=jnp.float32)
    y_ref[...] = xw * dinv
    dinv_ref[...] = jnp.broadcast_to(dinv, (RB, 16))


def _matmul1(x, W1, cnt):
    nb = N // RB
    return pl.pallas_call(
        _mm1_body,
        grid=(nb, 2),
        in_specs=[
            pl.BlockSpec((RB, D), lambda i, j: (i, 0)),
            pl.BlockSpec((D, H), lambda i, j: (0, j)),
            pl.BlockSpec((RB, 16), lambda i, j: (i, 0)),
            pl.BlockSpec((RB, 16), lambda i, j: (nb + i, 0)),
        ],
        out_specs=[
            pl.BlockSpec((RB, H), lambda i, j: (j * nb + i, 0)),
            pl.BlockSpec((RB, 16), lambda i, j: (i, 0)),
        ],
        out_shape=[
            jax.ShapeDtypeStruct((NC * N, H), jnp.float32),
            jax.ShapeDtypeStruct((N, 16), jnp.float32),
        ],
    )(x, W1, cnt, cnt)


def _layer2_body(a0_ref, a1_ref, y0_ref, y1_ref, cnt_ref, b1_ref, w_ref, o_ref):
    dinv = cnt_ref[:, 0:1]
    h0 = jnp.maximum((a0_ref[...] + y0_ref[...]) * dinv + b1_ref[0, 0:1, :], 0.0)
    h1 = jnp.maximum((a1_ref[...] + y1_ref[...]) * dinv + b1_ref[1, 0:1, :], 0.0)
    z = jnp.dot(h0, w_ref[0:H, :], preferred_element_type=jnp.float32)
    z = z + jnp.dot(h1, w_ref[H:D, :], preferred_element_type=jnp.float32)
    o_ref[...] = z * dinv


def _layer2(acc1, y1, cnt, b1p, W2):
    nb = N // RB
    return pl.pallas_call(
        _layer2_body,
        grid=(nb, 2),
        in_specs=[
            pl.BlockSpec((RB, H), lambda i, j: (i, 0)),
            pl.BlockSpec((RB, H), lambda i, j: (nb + i, 0)),
            pl.BlockSpec((RB, H), lambda i, j: (i, 0)),
            pl.BlockSpec((RB, H), lambda i, j: (nb + i, 0)),
            pl.BlockSpec((RB, 16), lambda i, j: (i, 0)),
            pl.BlockSpec((2, 8, H), lambda i, j: (0, 0, 0)),
            pl.BlockSpec((D, H), lambda i, j: (0, j)),
        ],
        out_specs=pl.BlockSpec((RB, H), lambda i, j: (j * nb + i, 0)),
        out_shape=jax.ShapeDtypeStruct((NC * N, H), jnp.float32),
    )(acc1, acc1, y1, y1, cnt, b1p, W2)


def _final_body(a_ref, y_ref, cnt_ref, b2_ref, o_ref):
    j = pl.program_id(1)
    dinv = cnt_ref[:, 0:1]
    o_ref[...] = (a_ref[...] + y_ref[...]) * dinv + b2_ref[j, 0:1, :]


def _final(acc2, yz, cnt, b2p):
    nb = N // RB
    return pl.pallas_call(
        _final_body,
        grid=(nb, 2),
        in_specs=[
            pl.BlockSpec((RB, H), lambda i, j: (j * nb + i, 0)),
            pl.BlockSpec((RB, H), lambda i, j: (j * nb + i, 0)),
            pl.BlockSpec((RB, 16), lambda i, j: (i, 0)),
            pl.BlockSpec((2, 8, H), lambda i, j: (0, 0, 0)),
        ],
        out_specs=pl.BlockSpec((RB, H), lambda i, j: (i, j)),
        out_shape=jax.ShapeDtypeStruct((N, D), jnp.float32),
    )(acc2, yz, cnt, b2p)


def kernel(x, edge_index, W1, b1, W2, b2):
    ei = edge_index.astype(jnp.int32)
    pad = EP - E
    src_p = jnp.concatenate([ei[0], jnp.zeros((pad,), jnp.int32)])
    dst_p = jnp.concatenate([ei[1], jnp.full((pad,), N, jnp.int32)])
    b1p = jnp.broadcast_to(b1.reshape(2, 1, H), (2, 8, H))
    b2p = jnp.broadcast_to(b2.reshape(2, 1, H), (2, 8, H))

    cnt = _deg_count(dst_p)
    y1, dinv16 = _matmul1(x, W1, cnt)
    acc1 = _edge_pass(y1, src_p, dst_p)
    yz = _layer2(acc1, y1, dinv16, b1p, W2)
    acc2 = _edge_pass(yz, src_p, dst_p)
    return _final(acc2, yz, dinv16, b2p)
